# baseline (speedup 1.0000x reference)
import functools

import jax
import jax.numpy as jnp
from jax import lax
from jax.experimental import pallas as pl
from jax.experimental.pallas import tpu as pltpu
from jax.experimental.pallas import tpu_sc as plsc

_B_GROUP = 4


@functools.lru_cache(maxsize=None)
def _make_gather(batch: int, seq: int, dim: int):
    info = plsc.get_sparse_core_info()
    nc, ns = info.num_cores, info.num_subcores
    nw = nc * ns
    assert batch % (nw * _B_GROUP) == 0 and seq % 2 == 0
    b_per_w = batch // nw
    splits = []
    off = 0
    while seq - off > 128:
        splits.append((off, 104))
        off += 104
    splits.append((off, seq - off))
    assert all(sz % 8 == 0 and sz <= 128 for _, sz in splits)
    n_groups = b_per_w // _B_GROUP

    mesh = plsc.VectorSubcoreMesh(core_axis_name="c", subcore_axis_name="s")

    @functools.partial(
        pl.kernel,
        mesh=mesh,
        out_type=jax.ShapeDtypeStruct((batch, seq, dim), jnp.float32),
        scratch_types=[
            pltpu.VMEM((b_per_w, seq), jnp.int32),
            pltpu.VMEM((2, _B_GROUP, seq, dim), jnp.float32),
            pltpu.VMEM((16,), jnp.int32),
            pltpu.SemaphoreType.DMA,
            pltpu.SemaphoreType.DMA,
            pltpu.SemaphoreType.DMA,
        ],
        compiler_params=pltpu.CompilerParams(use_tc_tiling_on_sc=False),
    )
    def gather_kernel(text_hbm, aim_hbm, table_hbm, out_hbm,
                      idx_v, rows_v, aim_v, sg0, sg1, sw):
        wid = lax.axis_index("s") * nc + lax.axis_index("c")
        row0 = wid * b_per_w
        pltpu.sync_copy(aim_hbm, aim_v)
        pltpu.sync_copy(text_hbm.at[pl.ds(row0, b_per_w)], idx_v)
        aim = aim_v[...][0]

        @pl.when(aim < seq)
        def _mask():
            lane = lax.iota(jnp.int32, 16)
            starts = list(range(0, seq - 15, 16))
            if starts[-1] != seq - 16:
                starts.append(seq - 16)

            def mask_row(r, carry):
                for c in starts:
                    v = idx_v[r, pl.ds(c, 16)]
                    idx_v[r, pl.ds(c, 16)] = jnp.where(lane + c < aim, v, 0)
                return carry

            lax.fori_loop(0, b_per_w, mask_row, 0)

        sgs = (sg0, sg1)

        def fire_gathers(g):
            buf = g % 2
            copies = []
            for lb in range(_B_GROUP):
                for off, sz in splits:
                    copies.append(pltpu.async_copy(
                        table_hbm.at[idx_v.at[g * _B_GROUP + lb,
                                              pl.ds(off, sz)]],
                        rows_v.at[buf, lb, pl.ds(off, sz)],
                        sgs[buf]))
            return copies

        pend = fire_gathers(0)
        wbs = []
        for g in range(n_groups):
            if g + 1 < n_groups:
                if g >= 1:
                    wbs[g - 1].wait()
                nxt = fire_gathers(g + 1)
            else:
                nxt = None
            for c in pend:
                c.wait()
            wbs.append(pltpu.async_copy(
                rows_v.at[g % 2],
                out_hbm.at[pl.ds(row0 + g * _B_GROUP, _B_GROUP)],
                sw))
            pend = nxt
        wbs[n_groups - 2].wait()
        wbs[n_groups - 1].wait()

    return gather_kernel


def kernel(text_bt, aim_seq_len, table):
    b, s = text_bt.shape
    dim = table.shape[1]
    aim_arr = jnp.broadcast_to(
        jnp.asarray(aim_seq_len, jnp.int32).reshape(1), (16,))
    return _make_gather(b, s, dim)(text_bt, aim_arr, table)
